# Initial kernel scaffold; baseline (speedup 1.0000x reference)
import jax
import jax.numpy as jnp
from jax.experimental import pallas as pl


def kernel(z_v, w1t_wide, b1_wide, w2t_wide, b2_wide):
    raise NotImplementedError("write your pallas kernel here")


# speedup vs baseline: 1.1079x; 1.1079x over previous
from functools import partial

import jax
import jax.numpy as jnp
from jax.experimental import pallas as pl
from jax.experimental.pallas import tpu as pltpu


def _round_up(x, m):
    return ((x + m - 1) // m) * m


def _mlp_kernel(z_ref, w1_ref, b1_ref, w2_ref, b2_ref, out_ref):
    z = z_ref[...].astype(jnp.bfloat16)
    h = jnp.dot(z, w1_ref[...], preferred_element_type=jnp.float32) + b1_ref[...]
    h = jnp.maximum(h, 0.0).astype(jnp.bfloat16)
    o = jnp.dot(h, w2_ref[...], preferred_element_type=jnp.float32) + b2_ref[...]
    out_ref[...] = o


@partial(jax.jit, static_argnames=("tile_rows",))
def _forward(z_v, w1t_wide, b1_wide, w2t_wide, b2_wide, *, tile_rows=1024):
    z_num, z_dim = z_v.shape
    zw = w1t_wide.shape[0]
    pack = zw // z_dim
    ow = w2t_wide.shape[1]
    out_features = ow // pack

    mp = pl.cdiv(z_num, pack)
    tile = min(tile_rows, _round_up(mp, 8))
    mp_pad = _round_up(mp, tile)
    grid = (mp_pad // tile,)

    flat = z_v.astype(jnp.float32).reshape(-1)
    pad = mp_pad * zw - flat.shape[0]
    if pad:
        flat = jnp.concatenate([flat, jnp.zeros((pad,), jnp.float32)])
    z_wide = flat.reshape(mp_pad, zw)

    w1b = w1t_wide.astype(jnp.bfloat16)
    w2b = w2t_wide.astype(jnp.bfloat16)

    hidden_w = w1t_wide.shape[1]
    cost = pl.CostEstimate(
        flops=2 * mp_pad * (zw * hidden_w + hidden_w * ow),
        transcendentals=0,
        bytes_accessed=4 * (mp_pad * (zw + ow)) + 2 * (w1b.size + w2b.size)
        + 4 * (b1_wide.size + b2_wide.size),
    )

    out_wide = pl.pallas_call(
        _mlp_kernel,
        out_shape=jax.ShapeDtypeStruct((mp_pad, ow), jnp.float32),
        grid=grid,
        in_specs=[
            pl.BlockSpec((tile, zw), lambda i: (i, 0)),
            pl.BlockSpec(w1b.shape, lambda i: (0, 0)),
            pl.BlockSpec(b1_wide.shape, lambda i: (0, 0)),
            pl.BlockSpec(w2b.shape, lambda i: (0, 0)),
            pl.BlockSpec(b2_wide.shape, lambda i: (0, 0)),
        ],
        out_specs=pl.BlockSpec((tile, ow), lambda i: (i, 0)),
        compiler_params=pltpu.CompilerParams(
            dimension_semantics=("parallel",),
            vmem_limit_bytes=64 * 1024 * 1024,
        ),
        cost_estimate=cost,
    )(z_wide, w1b, b1_wide, w2b, b2_wide)

    return out_wide.reshape(mp_pad * pack, out_features)[:z_num]


def kernel(z_v, w1t_wide, b1_wide, w2t_wide, b2_wide):
    return _forward(z_v, w1t_wide, b1_wide, w2t_wide, b2_wide)


# speedup vs baseline: 1.3768x; 1.2427x over previous
from functools import partial

import jax
import jax.numpy as jnp
from jax.experimental import pallas as pl
from jax.experimental.pallas import tpu as pltpu


def _round_up(x, m):
    return ((x + m - 1) // m) * m


def _mlp_kernel(z_ref, w1_ref, b1_ref, w2_ref, b2_ref, out_ref):
    h = jnp.dot(z_ref[...], w1_ref[...], preferred_element_type=jnp.float32) + b1_ref[...]
    h = jnp.maximum(h, 0.0)
    o = jnp.dot(h, w2_ref[...], preferred_element_type=jnp.float32) + b2_ref[...]
    out_ref[...] = o


@partial(jax.jit, static_argnames=("tile_rows",))
def _forward(z_v, w1t_wide, b1_wide, w2t_wide, b2_wide, *, tile_rows=4096):
    z_num, z_dim = z_v.shape
    pack = w1t_wide.shape[0] // z_dim
    hidden = w1t_wide.shape[1] // pack
    out_features = w2t_wide.shape[1] // pack

    w1t = w1t_wide[:z_dim, :hidden]
    b1 = b1_wide[:, :hidden]
    w2t = w2t_wide[:hidden, :out_features]
    b2 = b2_wide[:, :out_features]

    tile = min(tile_rows, _round_up(z_num, 8))
    n_pad = _round_up(z_num, tile)
    if n_pad != z_num:
        z_v = jnp.pad(z_v, ((0, n_pad - z_num), (0, 0)))
    grid = (n_pad // tile,)

    cost = pl.CostEstimate(
        flops=2 * n_pad * (z_dim * hidden + hidden * out_features),
        transcendentals=0,
        bytes_accessed=4 * (n_pad * (z_dim + out_features)
                            + w1t.size + b1.size + w2t.size + b2.size),
    )

    out = pl.pallas_call(
        _mlp_kernel,
        out_shape=jax.ShapeDtypeStruct((n_pad, out_features), jnp.float32),
        grid=grid,
        in_specs=[
            pl.BlockSpec((tile, z_dim), lambda i: (i, 0)),
            pl.BlockSpec(w1t.shape, lambda i: (0, 0)),
            pl.BlockSpec(b1.shape, lambda i: (0, 0)),
            pl.BlockSpec(w2t.shape, lambda i: (0, 0)),
            pl.BlockSpec(b2.shape, lambda i: (0, 0)),
        ],
        out_specs=pl.BlockSpec((tile, out_features), lambda i: (i, 0)),
        compiler_params=pltpu.CompilerParams(
            dimension_semantics=("parallel",),
            vmem_limit_bytes=64 * 1024 * 1024,
        ),
        cost_estimate=cost,
    )(z_v, w1t, b1, w2t, b2)

    return out[:z_num]


def kernel(z_v, w1t_wide, b1_wide, w2t_wide, b2_wide):
    return _forward(z_v, w1t_wide, b1_wide, w2t_wide, b2_wide)


# speedup vs baseline: 10.1729x; 7.3889x over previous
from functools import partial

import jax
import jax.numpy as jnp
from jax.experimental import pallas as pl
from jax.experimental.pallas import tpu as pltpu


def _mlp_kernel(zt_ref, w1_ref, b1_ref, w2_ref, b2_ref, out_ref):
    h = jnp.dot(w1_ref[...], zt_ref[...], preferred_element_type=jnp.float32) + b1_ref[...]
    h = jnp.maximum(h, 0.0)
    o = jnp.dot(w2_ref[...], h, preferred_element_type=jnp.float32) + b2_ref[...]
    out_ref[...] = o


@partial(jax.jit, static_argnames=("tile_cols",))
def _forward(z_v, w1t_wide, b1_wide, w2t_wide, b2_wide, *, tile_cols=16384):
    z_num, z_dim = z_v.shape
    pack = w1t_wide.shape[0] // z_dim
    hidden = w1t_wide.shape[1] // pack
    out_features = w2t_wide.shape[1] // pack

    zt = z_v.T

    w1 = w1t_wide[:z_dim, :hidden].T
    b1 = b1_wide[:1, :hidden].T
    w2 = w2t_wide[:hidden, :out_features].T
    b2 = b2_wide[:1, :out_features].T

    grid = (pl.cdiv(z_num, tile_cols),)

    cost = pl.CostEstimate(
        flops=2 * z_num * (z_dim * hidden + hidden * out_features),
        transcendentals=0,
        bytes_accessed=4 * (z_num * (z_dim + out_features)
                            + w1.size + b1.size + w2.size + b2.size),
    )

    out_t = pl.pallas_call(
        _mlp_kernel,
        out_shape=jax.ShapeDtypeStruct((out_features, z_num), jnp.float32),
        grid=grid,
        in_specs=[
            pl.BlockSpec((z_dim, tile_cols), lambda i: (0, i)),
            pl.BlockSpec(w1.shape, lambda i: (0, 0)),
            pl.BlockSpec(b1.shape, lambda i: (0, 0)),
            pl.BlockSpec(w2.shape, lambda i: (0, 0)),
            pl.BlockSpec(b2.shape, lambda i: (0, 0)),
        ],
        out_specs=pl.BlockSpec((out_features, tile_cols), lambda i: (0, i)),
        compiler_params=pltpu.CompilerParams(
            dimension_semantics=("parallel",),
            vmem_limit_bytes=64 * 1024 * 1024,
        ),
        cost_estimate=cost,
    )(zt, w1, b1, w2, b2)

    return out_t.T


def kernel(z_v, w1t_wide, b1_wide, w2t_wide, b2_wide):
    return _forward(z_v, w1t_wide, b1_wide, w2t_wide, b2_wide)


# speedup vs baseline: 10.7612x; 1.0578x over previous
from functools import partial

import jax
import jax.numpy as jnp
from jax.experimental import pallas as pl
from jax.experimental.pallas import tpu as pltpu


def _mlp_kernel(zt_ref, w1_ref, b1_ref, w2_ref, b2_ref, out_ref):
    h = jnp.dot(w1_ref[...], zt_ref[...], preferred_element_type=jnp.float32) + b1_ref[...]
    h = jnp.maximum(h, 0.0)
    o = jnp.dot(w2_ref[...], h, preferred_element_type=jnp.float32) + b2_ref[...]
    out_ref[...] = o


@partial(jax.jit, static_argnames=("tile_cols",))
def _forward(z_v, w1t_wide, b1_wide, w2t_wide, b2_wide, *, tile_cols=32768):
    z_num, z_dim = z_v.shape
    pack = w1t_wide.shape[0] // z_dim
    hidden = w1t_wide.shape[1] // pack
    out_features = w2t_wide.shape[1] // pack

    zt = z_v.T

    w1 = w1t_wide[:z_dim, :hidden].T
    b1 = b1_wide[:1, :hidden].T
    w2 = w2t_wide[:hidden, :out_features].T
    b2 = b2_wide[:1, :out_features].T

    grid = (pl.cdiv(z_num, tile_cols),)

    cost = pl.CostEstimate(
        flops=2 * z_num * (z_dim * hidden + hidden * out_features),
        transcendentals=0,
        bytes_accessed=4 * (z_num * (z_dim + out_features)
                            + w1.size + b1.size + w2.size + b2.size),
    )

    out_t = pl.pallas_call(
        _mlp_kernel,
        out_shape=jax.ShapeDtypeStruct((out_features, z_num), jnp.float32),
        grid=grid,
        in_specs=[
            pl.BlockSpec((z_dim, tile_cols), lambda i: (0, i)),
            pl.BlockSpec(w1.shape, lambda i: (0, 0)),
            pl.BlockSpec(b1.shape, lambda i: (0, 0)),
            pl.BlockSpec(w2.shape, lambda i: (0, 0)),
            pl.BlockSpec(b2.shape, lambda i: (0, 0)),
        ],
        out_specs=pl.BlockSpec((out_features, tile_cols), lambda i: (0, i)),
        compiler_params=pltpu.CompilerParams(
            dimension_semantics=("parallel",),
            vmem_limit_bytes=64 * 1024 * 1024,
        ),
        cost_estimate=cost,
    )(zt, w1, b1, w2, b2)

    return out_t.T


def kernel(z_v, w1t_wide, b1_wide, w2t_wide, b2_wide):
    return _forward(z_v, w1t_wide, b1_wide, w2t_wide, b2_wide)


# speedup vs baseline: 11.0486x; 1.0267x over previous
from functools import partial

import jax
import jax.numpy as jnp
from jax.experimental import pallas as pl
from jax.experimental.pallas import tpu as pltpu


def _mlp_kernel(zt_ref, w1_ref, b1_ref, w2_ref, b2_ref, out_ref):
    h = jnp.dot(w1_ref[...], zt_ref[...], preferred_element_type=jnp.float32) + b1_ref[...]
    h = jnp.maximum(h, 0.0)
    o = jnp.dot(w2_ref[...], h, preferred_element_type=jnp.float32) + b2_ref[...]
    out_ref[...] = o


@partial(jax.jit, static_argnames=("tile_cols",))
def _forward(z_v, w1t_wide, b1_wide, w2t_wide, b2_wide, *, tile_cols=65536):
    z_num, z_dim = z_v.shape
    pack = w1t_wide.shape[0] // z_dim
    hidden = w1t_wide.shape[1] // pack
    out_features = w2t_wide.shape[1] // pack

    zt = z_v.T

    w1 = w1t_wide[:z_dim, :hidden].T
    b1 = b1_wide[:1, :hidden].T
    w2 = w2t_wide[:hidden, :out_features].T
    b2 = b2_wide[:1, :out_features].T

    grid = (pl.cdiv(z_num, tile_cols),)

    cost = pl.CostEstimate(
        flops=2 * z_num * (z_dim * hidden + hidden * out_features),
        transcendentals=0,
        bytes_accessed=4 * (z_num * (z_dim + out_features)
                            + w1.size + b1.size + w2.size + b2.size),
    )

    out_t = pl.pallas_call(
        _mlp_kernel,
        out_shape=jax.ShapeDtypeStruct((out_features, z_num), jnp.float32),
        grid=grid,
        in_specs=[
            pl.BlockSpec((z_dim, tile_cols), lambda i: (0, i)),
            pl.BlockSpec(w1.shape, lambda i: (0, 0)),
            pl.BlockSpec(b1.shape, lambda i: (0, 0)),
            pl.BlockSpec(w2.shape, lambda i: (0, 0)),
            pl.BlockSpec(b2.shape, lambda i: (0, 0)),
        ],
        out_specs=pl.BlockSpec((out_features, tile_cols), lambda i: (0, i)),
        compiler_params=pltpu.CompilerParams(
            dimension_semantics=("parallel",),
            vmem_limit_bytes=64 * 1024 * 1024,
        ),
        cost_estimate=cost,
    )(zt, w1, b1, w2, b2)

    return out_t.T


def kernel(z_v, w1t_wide, b1_wide, w2t_wide, b2_wide):
    return _forward(z_v, w1t_wide, b1_wide, w2t_wide, b2_wide)
